# baseline (speedup 1.0000x reference)
import functools

import jax
import jax.numpy as jnp
from jax import lax
from jax.experimental import pallas as pl
from jax.experimental.pallas import tpu as pltpu
from jax.experimental.pallas import tpu_sc as plsc

T = 2048
H = 768
E = 8
K = 2
MI = 1024
ISH = 2048

BT = 256
NBLK = T * K // BT + E
NBLK_PAD = 32
PADN = NBLK * BT

NC, NS, NW, L = 2, 16, 32, 16


def _router_body(x_ref, wg_ref, logits_ref, topw_ref, slots_ref, eid_ref):
    x = x_ref[...]
    wg = wg_ref[...]
    logits = lax.dot_general(x, wg, (((1,), (1,)), ((), ())),
                             preferred_element_type=jnp.float32)
    logits_ref[...] = logits
    m = jnp.max(logits, axis=1, keepdims=True)
    ex = jnp.exp(logits - m)
    rw = ex / jnp.sum(ex, axis=1, keepdims=True)
    iota_e = lax.broadcasted_iota(jnp.int32, (T, E), 1)
    m0 = jnp.max(rw, axis=1, keepdims=True)
    i0 = jnp.min(jnp.where(rw == m0, iota_e, E), axis=1, keepdims=True)
    rw1 = jnp.where(iota_e == i0, -1.0, rw)
    m1 = jnp.max(rw1, axis=1, keepdims=True)
    i1 = jnp.min(jnp.where(rw1 == m1, iota_e, E), axis=1, keepdims=True)
    topw_ref[...] = jnp.concatenate([m0, m1], axis=1)
    oh0 = (iota_e == i0).astype(jnp.float32)
    oh1 = (iota_e == i1).astype(jnp.float32)
    cnt = oh0 + oh1
    s = cnt
    sh = 1
    while sh < T:
        s = s + jnp.concatenate(
            [jnp.zeros((sh, E), jnp.float32), s[:-sh, :]], axis=0)
        sh *= 2
    pre = s - cnt
    tot = s[T - 1:T, :]
    nb = jnp.floor((tot + (BT - 1)) / BT)
    ends = nb
    sh = 1
    while sh < E:
        ends = ends + jnp.concatenate(
            [jnp.zeros((1, sh), jnp.float32), ends[:, :-sh]], axis=1)
        sh *= 2
    offrow = (ends - nb) * float(BT)
    slot0 = jnp.sum(oh0 * (offrow + pre), axis=1, keepdims=True)
    slot1 = jnp.sum(oh1 * (offrow + pre), axis=1, keepdims=True)
    slots_ref[...] = jnp.concatenate([slot0, slot1], axis=1).astype(jnp.int32)
    bio = lax.broadcasted_iota(jnp.int32, (NBLK_PAD, E), 0).astype(jnp.float32)
    ge = (bio >= jnp.broadcast_to(ends, (NBLK_PAD, E))).astype(jnp.float32)
    eidf = jnp.minimum(jnp.sum(ge, axis=1, keepdims=True), float(E - 1))
    eid_ref[...] = eidf.astype(jnp.int32)


_router = pl.pallas_call(
    _router_body,
    out_shape=[
        jax.ShapeDtypeStruct((T, E), jnp.float32),
        jax.ShapeDtypeStruct((T, K), jnp.float32),
        jax.ShapeDtypeStruct((T, K), jnp.int32),
        jax.ShapeDtypeStruct((NBLK_PAD, 1), jnp.int32),
    ],
)


def _sc_scatter_body(p0_hbm, p1_hbm, w0_hbm, w1_hbm, src_hbm, wsl_hbm,
                     p0_v, p1_v, w0_v, w1_v, src_v, wsl_v):
    wid = lax.axis_index("s") * NC + lax.axis_index("c")

    @pl.when(wid == 0)
    def _():
        pltpu.sync_copy(p0_hbm, p0_v)
        pltpu.sync_copy(p1_hbm, p1_v)
        pltpu.sync_copy(w0_hbm, w0_v)
        pltpu.sync_copy(w1_hbm, w1_v)

        @pl.loop(0, PADN // L)
        def _(i):
            src_v[pl.ds(i * L, L)] = jnp.zeros((L,), jnp.int32)
            wsl_v[pl.ds(i * L, L)] = jnp.zeros((L,), jnp.float32)

        @pl.loop(0, T // L)
        def _(i):
            base = i * L
            tvec = lax.iota(jnp.int32, L) + base
            idx0 = p0_v[pl.ds(base, L)]
            idx1 = p1_v[pl.ds(base, L)]
            plsc.store_scatter(src_v, [idx0], tvec)
            plsc.store_scatter(wsl_v, [idx0], w0_v[pl.ds(base, L)])
            plsc.store_scatter(src_v, [idx1], tvec)
            plsc.store_scatter(wsl_v, [idx1], w1_v[pl.ds(base, L)])

        pltpu.sync_copy(src_v, src_hbm)
        pltpu.sync_copy(wsl_v, wsl_hbm)


RPW = PADN // NW
RCH = 96


def _sc_gather_body(x_hbm, src_hbm, xs_hbm, idx_v, rows_v, sem):
    wid = lax.axis_index("s") * NC + lax.axis_index("c")
    base = wid * RPW
    for r in range(RPW // RCH):
        off = base + r * RCH
        pltpu.sync_copy(src_hbm.at[pl.ds(off, RCH)], idx_v)
        pltpu.async_copy(x_hbm.at[idx_v], rows_v, sem).wait()
        pltpu.sync_copy(rows_v, xs_hbm.at[pl.ds(off, RCH)])


def _moe_body(eid_ref, xs_ref, weg_ref, weu_ref, wed_ref, wsl_ref, out_ref):
    del eid_ref
    xb = xs_ref[...]
    wg = weg_ref[0]
    wu = weu_ref[0]
    wd = wed_ref[0]
    g = lax.dot_general(xb, wg, (((1,), (1,)), ((), ())),
                        preferred_element_type=jnp.float32)
    u = lax.dot_general(xb, wu, (((1,), (1,)), ((), ())),
                        preferred_element_type=jnp.float32)
    h = (g * jax.nn.sigmoid(g)) * u
    o = lax.dot_general(h, wd, (((1,), (1,)), ((), ())),
                        preferred_element_type=jnp.float32)
    out_ref[...] = o * wsl_ref[0]


_moe_mm = pl.pallas_call(
    _moe_body,
    grid_spec=pltpu.PrefetchScalarGridSpec(
        num_scalar_prefetch=1,
        grid=(NBLK,),
        in_specs=[
            pl.BlockSpec((BT, H), lambda b, eid: (b, 0)),
            pl.BlockSpec((1, MI, H), lambda b, eid: (eid[b], 0, 0)),
            pl.BlockSpec((1, MI, H), lambda b, eid: (eid[b], 0, 0)),
            pl.BlockSpec((1, H, MI), lambda b, eid: (eid[b], 0, 0)),
            pl.BlockSpec((1, BT, 1), lambda b, eid: (b, 0, 0)),
        ],
        out_specs=pl.BlockSpec((BT, H), lambda b, eid: (b, 0)),
    ),
    out_shape=jax.ShapeDtypeStruct((PADN, H), jnp.float32),
)


TPW = T // NW


def _sc_combine_body(eo_hbm, p0_hbm, p1_hbm, moe_hbm,
                     i0_v, i1_v, r0, r1, sem0, sem1):
    wid = lax.axis_index("s") * NC + lax.axis_index("c")
    base = wid * TPW
    pltpu.sync_copy(p0_hbm.at[pl.ds(base, TPW)], i0_v)
    pltpu.sync_copy(p1_hbm.at[pl.ds(base, TPW)], i1_v)
    c0 = pltpu.async_copy(eo_hbm.at[i0_v], r0, sem0)
    c1 = pltpu.async_copy(eo_hbm.at[i1_v], r1, sem1)
    c0.wait()
    c1.wait()

    @pl.loop(0, TPW)
    def _(j):
        @pl.loop(0, H // L)
        def _(cchunk):
            sl = pl.ds(cchunk * L, L)
            r0[j, sl] = r0[j, sl] + r1[j, sl]

    pltpu.sync_copy(r0, moe_hbm.at[pl.ds(base, TPW)])


@functools.lru_cache(maxsize=1)
def _build_sc_kernels():
    mesh = plsc.VectorSubcoreMesh(core_axis_name="c", subcore_axis_name="s")
    sc_scatter = functools.partial(
        pl.kernel,
        out_type=[jax.ShapeDtypeStruct((PADN,), jnp.int32),
                  jax.ShapeDtypeStruct((PADN,), jnp.float32)],
        mesh=mesh,
        scratch_types=[
            pltpu.VMEM((T,), jnp.int32), pltpu.VMEM((T,), jnp.int32),
            pltpu.VMEM((T,), jnp.float32), pltpu.VMEM((T,), jnp.float32),
            pltpu.VMEM((PADN,), jnp.int32), pltpu.VMEM((PADN,), jnp.float32),
        ],
        compiler_params=pltpu.CompilerParams(needs_layout_passes=False),
    )(_sc_scatter_body)
    sc_gather = functools.partial(
        pl.kernel,
        out_type=jax.ShapeDtypeStruct((PADN, H), jnp.float32),
        mesh=mesh,
        scratch_types=[pltpu.VMEM((RCH,), jnp.int32),
                       pltpu.VMEM((RCH, H), jnp.float32),
                       pltpu.SemaphoreType.DMA],
    )(_sc_gather_body)
    sc_combine = functools.partial(
        pl.kernel,
        out_type=jax.ShapeDtypeStruct((T, H), jnp.float32),
        mesh=mesh,
        scratch_types=[pltpu.VMEM((TPW,), jnp.int32),
                       pltpu.VMEM((TPW,), jnp.int32),
                       pltpu.VMEM((TPW, H), jnp.float32),
                       pltpu.VMEM((TPW, H), jnp.float32),
                       pltpu.SemaphoreType.DMA, pltpu.SemaphoreType.DMA],
    )(_sc_combine_body)
    return sc_scatter, sc_gather, sc_combine


BTF = 256


def _final_body(x_ref, moe_ref, wsg_ref, wsu_ref, wsd_ref, wsig_ref, out_ref):
    xb = x_ref[...]
    g = lax.dot_general(xb, wsg_ref[...], (((1,), (1,)), ((), ())),
                        preferred_element_type=jnp.float32)
    u = lax.dot_general(xb, wsu_ref[...], (((1,), (1,)), ((), ())),
                        preferred_element_type=jnp.float32)
    h = (g * jax.nn.sigmoid(g)) * u
    shd = lax.dot_general(h, wsd_ref[...], (((1,), (1,)), ((), ())),
                          preferred_element_type=jnp.float32)
    sg = jax.nn.sigmoid(
        lax.dot_general(xb, wsig_ref[...], (((1,), (1,)), ((), ())),
                        preferred_element_type=jnp.float32))
    out_ref[...] = moe_ref[...] + sg * shd


_final = pl.pallas_call(
    _final_body,
    grid=(T // BTF,),
    in_specs=[
        pl.BlockSpec((BTF, H), lambda b: (b, 0)),
        pl.BlockSpec((BTF, H), lambda b: (b, 0)),
        pl.BlockSpec((ISH, H), lambda b: (0, 0)),
        pl.BlockSpec((ISH, H), lambda b: (0, 0)),
        pl.BlockSpec((H, ISH), lambda b: (0, 0)),
        pl.BlockSpec((1, H), lambda b: (0, 0)),
    ],
    out_specs=pl.BlockSpec((BTF, H), lambda b: (b, 0)),
    out_shape=jax.ShapeDtypeStruct((T, H), jnp.float32),
)


def kernel(hidden_states, Wg, We_gate, We_up, We_down,
           Ws_gate, Ws_up, Ws_down, Wsg):
    b, s_, h = hidden_states.shape
    x = hidden_states.reshape(s_, h)
    logits, topw, slots, eid2 = _router(x, Wg)
    p0 = slots[:, 0]
    p1 = slots[:, 1]
    w0 = topw[:, 0]
    w1 = topw[:, 1]
    eid_arr = eid2.reshape(NBLK_PAD)[:NBLK]
    _sc_scatter, _sc_gather, _sc_combine = _build_sc_kernels()
    src_tok, wslot = _sc_scatter(p0, p1, w0, w1)
    xs = _sc_gather(x, src_tok)
    eo = _moe_mm(eid_arr, xs, We_gate, We_up, We_down,
                 wslot.reshape(NBLK, BT, 1))
    moe = _sc_combine(eo, p0, p1)
    final = _final(x, moe, Ws_gate, Ws_up, Ws_down, Wsg)
    return final.reshape(b, s_, h), logits


# speedup vs baseline: 1.0344x; 1.0344x over previous
import functools

import jax
import jax.numpy as jnp
from jax import lax
from jax.experimental import pallas as pl
from jax.experimental.pallas import tpu as pltpu
from jax.experimental.pallas import tpu_sc as plsc

T = 2048
H = 768
E = 8
K = 2
MI = 1024
ISH = 2048

BT = 256
NBLK = T * K // BT + E
NBLK_PAD = 32
PADN = NBLK * BT

NC, NS, NW, L = 2, 16, 32, 16


def _router_body(x_ref, wg_ref, logits_ref, topw_ref, slots_ref, eid_ref):
    x = x_ref[...]
    wg = wg_ref[...]
    logits = lax.dot_general(x, wg, (((1,), (1,)), ((), ())),
                             preferred_element_type=jnp.float32)
    logits_ref[...] = logits
    m = jnp.max(logits, axis=1, keepdims=True)
    ex = jnp.exp(logits - m)
    rw = ex / jnp.sum(ex, axis=1, keepdims=True)
    iota_e = lax.broadcasted_iota(jnp.int32, (T, E), 1)
    m0 = jnp.max(rw, axis=1, keepdims=True)
    i0 = jnp.min(jnp.where(rw == m0, iota_e, E), axis=1, keepdims=True)
    rw1 = jnp.where(iota_e == i0, -1.0, rw)
    m1 = jnp.max(rw1, axis=1, keepdims=True)
    i1 = jnp.min(jnp.where(rw1 == m1, iota_e, E), axis=1, keepdims=True)
    topw_ref[...] = jnp.concatenate([m0, m1], axis=1)
    oh0 = (iota_e == i0).astype(jnp.float32)
    oh1 = (iota_e == i1).astype(jnp.float32)
    cnt = oh0 + oh1
    s = cnt
    sh = 1
    while sh < T:
        s = s + jnp.concatenate(
            [jnp.zeros((sh, E), jnp.float32), s[:-sh, :]], axis=0)
        sh *= 2
    pre = s - cnt
    tot = s[T - 1:T, :]
    nb = jnp.floor((tot + (BT - 1)) / BT)
    ends = nb
    sh = 1
    while sh < E:
        ends = ends + jnp.concatenate(
            [jnp.zeros((1, sh), jnp.float32), ends[:, :-sh]], axis=1)
        sh *= 2
    offrow = (ends - nb) * float(BT)
    slot0 = jnp.sum(oh0 * (offrow + pre), axis=1, keepdims=True)
    slot1 = jnp.sum(oh1 * (offrow + pre), axis=1, keepdims=True)
    slots_ref[...] = jnp.concatenate([slot0, slot1], axis=1).astype(jnp.int32)
    bio = lax.broadcasted_iota(jnp.int32, (NBLK_PAD, E), 0).astype(jnp.float32)
    ge = (bio >= jnp.broadcast_to(ends, (NBLK_PAD, E))).astype(jnp.float32)
    eidf = jnp.minimum(jnp.sum(ge, axis=1, keepdims=True), float(E - 1))
    eid_ref[...] = eidf.astype(jnp.int32)


_router = pl.pallas_call(
    _router_body,
    out_shape=[
        jax.ShapeDtypeStruct((T, E), jnp.float32),
        jax.ShapeDtypeStruct((T, K), jnp.float32),
        jax.ShapeDtypeStruct((T, K), jnp.int32),
        jax.ShapeDtypeStruct((NBLK_PAD, 1), jnp.int32),
    ],
)


def _sc_scatter_body(p0_hbm, p1_hbm, w0_hbm, w1_hbm, src_hbm, wsl_hbm,
                     p0_v, p1_v, w0_v, w1_v, src_v, wsl_v):
    wid = lax.axis_index("s") * NC + lax.axis_index("c")

    @pl.when(wid == 0)
    def _():
        pltpu.sync_copy(p0_hbm, p0_v)
        pltpu.sync_copy(p1_hbm, p1_v)
        pltpu.sync_copy(w0_hbm, w0_v)
        pltpu.sync_copy(w1_hbm, w1_v)

        @pl.loop(0, PADN // L)
        def _(i):
            src_v[pl.ds(i * L, L)] = jnp.zeros((L,), jnp.int32)
            wsl_v[pl.ds(i * L, L)] = jnp.zeros((L,), jnp.float32)

        @pl.loop(0, T // L)
        def _(i):
            base = i * L
            tvec = lax.iota(jnp.int32, L) + base
            idx0 = p0_v[pl.ds(base, L)]
            idx1 = p1_v[pl.ds(base, L)]
            plsc.store_scatter(src_v, [idx0], tvec)
            plsc.store_scatter(wsl_v, [idx0], w0_v[pl.ds(base, L)])
            plsc.store_scatter(src_v, [idx1], tvec)
            plsc.store_scatter(wsl_v, [idx1], w1_v[pl.ds(base, L)])

        pltpu.sync_copy(src_v, src_hbm)
        pltpu.sync_copy(wsl_v, wsl_hbm)


RPW = PADN // NW
RCH = 64


def _sc_gather_body(x_hbm, src_hbm, xs_hbm, idx_v, ra, rb, sem_g, sw0, sw1):
    wid = lax.axis_index("s") * NC + lax.axis_index("c")
    base = wid * RPW
    pltpu.sync_copy(src_hbm.at[pl.ds(base, RPW)], idx_v)
    bufs = (ra, rb)
    wsems = (sw0, sw1)
    nr = RPW // RCH
    writes = [None] * nr
    for r in range(nr):
        buf = bufs[r % 2]
        if r >= 2:
            writes[r - 2].wait()
        idx_sl = idx_v.at[pl.ds(r * RCH, RCH)]
        pltpu.async_copy(x_hbm.at[idx_sl], buf, sem_g).wait()
        writes[r] = pltpu.async_copy(
            buf, xs_hbm.at[pl.ds(base + r * RCH, RCH)], wsems[r % 2])
    for r in range(max(nr - 2, 0), nr):
        writes[r].wait()


def _moe_body(eid_ref, xs_ref, weg_ref, weu_ref, wed_ref, wsl_ref, out_ref):
    del eid_ref
    xb = xs_ref[...]
    wg = weg_ref[0]
    wu = weu_ref[0]
    wd = wed_ref[0]
    g = lax.dot_general(xb, wg, (((1,), (1,)), ((), ())),
                        preferred_element_type=jnp.float32)
    u = lax.dot_general(xb, wu, (((1,), (1,)), ((), ())),
                        preferred_element_type=jnp.float32)
    h = (g * jax.nn.sigmoid(g)) * u
    o = lax.dot_general(h, wd, (((1,), (1,)), ((), ())),
                        preferred_element_type=jnp.float32)
    out_ref[...] = o * wsl_ref[0]


_moe_mm = pl.pallas_call(
    _moe_body,
    grid_spec=pltpu.PrefetchScalarGridSpec(
        num_scalar_prefetch=1,
        grid=(NBLK,),
        in_specs=[
            pl.BlockSpec((BT, H), lambda b, eid: (b, 0)),
            pl.BlockSpec((1, MI, H), lambda b, eid: (eid[b], 0, 0)),
            pl.BlockSpec((1, MI, H), lambda b, eid: (eid[b], 0, 0)),
            pl.BlockSpec((1, H, MI), lambda b, eid: (eid[b], 0, 0)),
            pl.BlockSpec((1, BT, 1), lambda b, eid: (b, 0, 0)),
        ],
        out_specs=pl.BlockSpec((BT, H), lambda b, eid: (b, 0)),
    ),
    out_shape=jax.ShapeDtypeStruct((PADN, H), jnp.float32),
)


TPW = T // NW


def _sc_combine_body(eo_hbm, p0_hbm, p1_hbm, moe_hbm,
                     i0_v, i1_v, r0, r1, sem0, sem1):
    wid = lax.axis_index("s") * NC + lax.axis_index("c")
    base = wid * TPW
    pltpu.sync_copy(p0_hbm.at[pl.ds(base, TPW)], i0_v)
    pltpu.sync_copy(p1_hbm.at[pl.ds(base, TPW)], i1_v)
    c0 = pltpu.async_copy(eo_hbm.at[i0_v], r0, sem0)
    c1 = pltpu.async_copy(eo_hbm.at[i1_v], r1, sem1)
    c0.wait()
    c1.wait()

    @pl.loop(0, TPW)
    def _(j):
        @pl.loop(0, H // L, unroll=8)
        def _(cchunk):
            sl = pl.ds(cchunk * L, L)
            r0[j, sl] = r0[j, sl] + r1[j, sl]

    pltpu.sync_copy(r0, moe_hbm.at[pl.ds(base, TPW)])


@functools.lru_cache(maxsize=1)
def _build_sc_kernels():
    mesh = plsc.VectorSubcoreMesh(core_axis_name="c", subcore_axis_name="s")
    sc_scatter = functools.partial(
        pl.kernel,
        out_type=[jax.ShapeDtypeStruct((PADN,), jnp.int32),
                  jax.ShapeDtypeStruct((PADN,), jnp.float32)],
        mesh=mesh,
        scratch_types=[
            pltpu.VMEM((T,), jnp.int32), pltpu.VMEM((T,), jnp.int32),
            pltpu.VMEM((T,), jnp.float32), pltpu.VMEM((T,), jnp.float32),
            pltpu.VMEM((PADN,), jnp.int32), pltpu.VMEM((PADN,), jnp.float32),
        ],
        compiler_params=pltpu.CompilerParams(needs_layout_passes=False),
    )(_sc_scatter_body)
    sc_gather = functools.partial(
        pl.kernel,
        out_type=jax.ShapeDtypeStruct((PADN, H), jnp.float32),
        mesh=mesh,
        scratch_types=[pltpu.VMEM((RPW,), jnp.int32),
                       pltpu.VMEM((RCH, H), jnp.float32),
                       pltpu.VMEM((RCH, H), jnp.float32),
                       pltpu.SemaphoreType.DMA,
                       pltpu.SemaphoreType.DMA,
                       pltpu.SemaphoreType.DMA],
    )(_sc_gather_body)
    sc_combine = functools.partial(
        pl.kernel,
        out_type=jax.ShapeDtypeStruct((T, H), jnp.float32),
        mesh=mesh,
        scratch_types=[pltpu.VMEM((TPW,), jnp.int32),
                       pltpu.VMEM((TPW,), jnp.int32),
                       pltpu.VMEM((TPW, H), jnp.float32),
                       pltpu.VMEM((TPW, H), jnp.float32),
                       pltpu.SemaphoreType.DMA, pltpu.SemaphoreType.DMA],
    )(_sc_combine_body)
    return sc_scatter, sc_gather, sc_combine


BTF = 256


def _shared_body(x_ref, wsg_ref, wsu_ref, wsd_ref, wsig_ref, out_ref):
    xb = x_ref[...]
    g = lax.dot_general(xb, wsg_ref[...], (((1,), (1,)), ((), ())),
                        preferred_element_type=jnp.float32)
    u = lax.dot_general(xb, wsu_ref[...], (((1,), (1,)), ((), ())),
                        preferred_element_type=jnp.float32)
    h = (g * jax.nn.sigmoid(g)) * u
    shd = lax.dot_general(h, wsd_ref[...], (((1,), (1,)), ((), ())),
                          preferred_element_type=jnp.float32)
    sg = jax.nn.sigmoid(
        lax.dot_general(xb, wsig_ref[...], (((1,), (1,)), ((), ())),
                        preferred_element_type=jnp.float32))
    out_ref[...] = sg * shd


_shared = pl.pallas_call(
    _shared_body,
    grid=(T // BTF,),
    in_specs=[
        pl.BlockSpec((BTF, H), lambda b: (b, 0)),
        pl.BlockSpec((ISH, H), lambda b: (0, 0)),
        pl.BlockSpec((ISH, H), lambda b: (0, 0)),
        pl.BlockSpec((H, ISH), lambda b: (0, 0)),
        pl.BlockSpec((1, H), lambda b: (0, 0)),
    ],
    out_specs=pl.BlockSpec((BTF, H), lambda b: (b, 0)),
    out_shape=jax.ShapeDtypeStruct((T, H), jnp.float32),
)


def _fadd_body(moe_ref, sgsh_ref, out_ref):
    out_ref[...] = moe_ref[...] + sgsh_ref[...]


_fadd = pl.pallas_call(
    _fadd_body,
    grid=(T // 512,),
    in_specs=[
        pl.BlockSpec((512, H), lambda b: (b, 0)),
        pl.BlockSpec((512, H), lambda b: (b, 0)),
    ],
    out_specs=pl.BlockSpec((512, H), lambda b: (b, 0)),
    out_shape=jax.ShapeDtypeStruct((T, H), jnp.float32),
)


def kernel(hidden_states, Wg, We_gate, We_up, We_down,
           Ws_gate, Ws_up, Ws_down, Wsg):
    b, s_, h = hidden_states.shape
    x = hidden_states.reshape(s_, h)
    logits, topw, slots, eid2 = _router(x, Wg)
    p0 = slots[:, 0]
    p1 = slots[:, 1]
    w0 = topw[:, 0]
    w1 = topw[:, 1]
    eid_arr = eid2.reshape(NBLK_PAD)[:NBLK]
    _sc_scatter, _sc_gather, _sc_combine = _build_sc_kernels()
    src_tok, wslot = _sc_scatter(p0, p1, w0, w1)
    sgsh = _shared(x, Ws_gate, Ws_up, Ws_down, Wsg)
    xs = _sc_gather(x, src_tok)
    eo = _moe_mm(eid_arr, xs, We_gate, We_up, We_down,
                 wslot.reshape(NBLK, BT, 1))
    moe = _sc_combine(eo, p0, p1)
    final = _fadd(moe, sgsh)
    return final.reshape(b, s_, h), logits
